# baseline (speedup 1.0000x reference)
import jax
import jax.numpy as jnp
from jax import lax
from jax.experimental import pallas as pl
from jax.experimental.pallas import tpu as pltpu
from jax.experimental.pallas import tpu_sc as plsc

N_NODES = 10000
N_PAD = 10240
D = 128
N_GRAPHS = 64
NC, NS = 2, 16
CHUNK = 128
GRP = 8
CPT = 80
NGRP = CPT // GRP
E_PAD = NC * NS * CPT * CHUNK
ROWS_PER_TILE = N_PAD // NS
BLK = 512
GRID = N_PAD // BLK

_MESH = plsc.VectorSubcoreMesh(core_axis_name="c", subcore_axis_name="s")


def _segsum_body(h_hbm, idx_hbm, zeros_hbm, out_hbm,
                 win_a, win_b, buf_a, buf_b, acc,
                 sem_ia, sem_ib, sem_ga, sem_gb, sem_sa, sem_sb):
  cid = lax.axis_index("c")
  sid = lax.axis_index("s")
  row0 = sid * ROWS_PER_TILE

  pltpu.sync_copy(zeros_hbm, acc.at[pl.ds(row0, ROWS_PER_TILE)])

  pltpu.async_copy(idx_hbm.at[cid, sid, pl.ds(0, GRP)], win_a, sem_ia)
  pltpu.async_copy(idx_hbm.at[cid, sid, pl.ds(GRP, GRP)], win_b, sem_ib)
  plsc.subcore_barrier()

  bufs = (buf_a, buf_b)
  gsems = (sem_ga, sem_gb)
  ssems = (sem_sa, sem_sb)

  def group(g, win, sem_i):
    pltpu.make_async_copy(idx_hbm.at[cid, sid, pl.ds(g * GRP, GRP)],
                          win, sem_i).wait()

    @pl.when(g > 0)
    def _():
      pltpu.make_async_copy(bufs[1], acc.at[win.at[GRP - 1, 1]],
                            ssems[1]).wait()

    pltpu.async_copy(h_hbm.at[win.at[0, 0]], bufs[0], gsems[0])
    for k in range(GRP):
      if k >= 1:
        pltpu.make_async_copy(bufs[(k - 1) % 2], acc.at[win.at[k - 1, 1]],
                              ssems[(k - 1) % 2]).wait()
      if k + 1 < GRP:
        pltpu.async_copy(h_hbm.at[win.at[k + 1, 0]],
                         bufs[(k + 1) % 2], gsems[(k + 1) % 2])
      pltpu.make_async_copy(h_hbm.at[win.at[k, 0]],
                            bufs[k % 2], gsems[k % 2]).wait()
      pltpu.async_copy(bufs[k % 2], acc.at[win.at[k, 1]], ssems[k % 2],
                       add=True)

    @pl.when(g + 2 < NGRP)
    def _():
      pltpu.async_copy(idx_hbm.at[cid, sid, pl.ds((g + 2) * GRP, GRP)],
                       win, sem_i)

  def pair(j, carry):
    group(2 * j, win_a, sem_ia)
    group(2 * j + 1, win_b, sem_ib)
    return carry

  lax.fori_loop(0, NGRP // 2, pair, 0)
  pltpu.make_async_copy(bufs[1], acc.at[win_b.at[GRP - 1, 1]],
                        ssems[1]).wait()
  plsc.subcore_barrier()

  pltpu.sync_copy(acc.at[pl.ds(row0, ROWS_PER_TILE)],
                  out_hbm.at[cid, pl.ds(row0, ROWS_PER_TILE)])


_sc_segsum = pl.kernel(
    _segsum_body,
    out_type=jax.ShapeDtypeStruct((NC, N_PAD, D), jnp.float32),
    mesh=_MESH,
    scratch_types=[
        pltpu.VMEM((GRP, 2, CHUNK), jnp.int32),
        pltpu.VMEM((GRP, 2, CHUNK), jnp.int32),
        pltpu.VMEM((CHUNK, D), jnp.float32),
        pltpu.VMEM((CHUNK, D), jnp.float32),
        pltpu.VMEM_SHARED((N_PAD, D), jnp.float32),
        pltpu.SemaphoreType.DMA,
        pltpu.SemaphoreType.DMA,
        pltpu.SemaphoreType.DMA,
        pltpu.SemaphoreType.DMA,
        pltpu.SemaphoreType.DMA,
        pltpu.SemaphoreType.DMA,
    ],
)


def _cnt_body(idx_hbm, zeros_hbm, ones_hbm, out_hbm, win_a, win_b, ones_v,
              acc, sem_ia, sem_ib):
  cid = lax.axis_index("c")
  sid = lax.axis_index("s")
  row0 = sid * ROWS_PER_TILE

  pltpu.sync_copy(ones_hbm, ones_v)
  pltpu.sync_copy(zeros_hbm, acc.at[pl.ds(row0, ROWS_PER_TILE)])
  pltpu.async_copy(idx_hbm.at[cid, sid, pl.ds(0, GRP)], win_a, sem_ia)
  pltpu.async_copy(idx_hbm.at[cid, sid, pl.ds(GRP, GRP)], win_b, sem_ib)
  plsc.subcore_barrier()

  def group(g, win, sem_i):
    pltpu.make_async_copy(idx_hbm.at[cid, sid, pl.ds(g * GRP, GRP)],
                          win, sem_i).wait()
    for k in range(GRP):
      pltpu.sync_copy(ones_v, acc.at[win.at[k, 1]], add=True)

    @pl.when(g + 2 < NGRP)
    def _():
      pltpu.async_copy(idx_hbm.at[cid, sid, pl.ds((g + 2) * GRP, GRP)],
                       win, sem_i)

  def pair(j, carry):
    group(2 * j, win_a, sem_ia)
    group(2 * j + 1, win_b, sem_ib)
    return carry

  lax.fori_loop(0, NGRP // 2, pair, 0)
  plsc.subcore_barrier()
  pltpu.sync_copy(acc.at[pl.ds(row0, ROWS_PER_TILE)],
                  out_hbm.at[cid, pl.ds(row0, ROWS_PER_TILE)])


_sc_cnt = pl.kernel(
    _cnt_body,
    out_type=jax.ShapeDtypeStruct((NC, N_PAD, D), jnp.float32),
    mesh=_MESH,
    scratch_types=[
        pltpu.VMEM((GRP, 2, CHUNK), jnp.int32),
        pltpu.VMEM((GRP, 2, CHUNK), jnp.int32),
        pltpu.VMEM((CHUNK, D), jnp.float32),
        pltpu.VMEM_SHARED((N_PAD, D), jnp.float32),
        pltpu.SemaphoreType.DMA,
        pltpu.SemaphoreType.DMA,
    ],
)


def _layer_body(p_ref, c_ref, x_ref, wl_ref, wr_ref, bl_ref, o_ref):
  i = pl.program_id(0)
  msg = p_ref[0] + p_ref[1]
  cnt = c_ref[0][:, 0:1] + c_ref[1][:, 0:1]
  inv = 1.0 / jnp.maximum(cnt, 1.0)
  h = inv * jnp.dot(msg, wl_ref[...], preferred_element_type=jnp.float32)
  h = h + bl_ref[...] + jnp.dot(x_ref[...], wr_ref[...],
                                preferred_element_type=jnp.float32)
  h = jnp.maximum(h, 0.0)
  rowid = i * BLK + lax.broadcasted_iota(jnp.int32, (BLK, 1), 0)
  o_ref[...] = jnp.where(rowid < N_NODES, h, 0.0)


_tc_layer = pl.pallas_call(
    _layer_body,
    grid=(GRID,),
    in_specs=[
        pl.BlockSpec((NC, BLK, D), lambda i: (0, i, 0)),
        pl.BlockSpec((NC, BLK, D), lambda i: (0, i, 0)),
        pl.BlockSpec((BLK, D), lambda i: (i, 0)),
        pl.BlockSpec((D, D), lambda i: (0, 0)),
        pl.BlockSpec((D, D), lambda i: (0, 0)),
        pl.BlockSpec((1, D), lambda i: (0, 0)),
    ],
    out_specs=pl.BlockSpec((BLK, D), lambda i: (i, 0)),
    out_shape=jax.ShapeDtypeStruct((N_PAD, D), jnp.float32),
)


def _final_body(h_ref, b_ref, g_ref, beta_ref, wd_ref, bd_ref, o_ref, pool):
  i = pl.program_id(0)

  @pl.when(i == 0)
  def _():
    pool[...] = jnp.zeros_like(pool)

  b = b_ref[...]
  onehot = (b == lax.broadcasted_iota(jnp.int32, (1, N_GRAPHS), 1)
            ).astype(jnp.float32)
  pool[...] += lax.dot_general(onehot, h_ref[...],
                               (((0,), (0,)), ((), ())),
                               preferred_element_type=jnp.float32)

  @pl.when(i == GRID - 1)
  def _():
    p = pool[...]
    m = jnp.mean(p, axis=1, keepdims=True)
    v = jnp.mean((p - m) * (p - m), axis=1, keepdims=True)
    ln = (p - m) * lax.rsqrt(v + 1e-5) * g_ref[...] + beta_ref[...]
    o_ref[...] = jnp.dot(ln, wd_ref[...],
                         preferred_element_type=jnp.float32) + bd_ref[...]


_tc_final = pl.pallas_call(
    _final_body,
    grid=(GRID,),
    in_specs=[
        pl.BlockSpec((BLK, D), lambda i: (i, 0)),
        pl.BlockSpec((BLK, 1), lambda i: (i, 0)),
        pl.BlockSpec((1, D), lambda i: (0, 0)),
        pl.BlockSpec((1, D), lambda i: (0, 0)),
        pl.BlockSpec((D, D), lambda i: (0, 0)),
        pl.BlockSpec((1, D), lambda i: (0, 0)),
    ],
    out_specs=pl.BlockSpec((N_GRAPHS, D), lambda i: (0, 0)),
    out_shape=jax.ShapeDtypeStruct((N_GRAPHS, D), jnp.float32),
    scratch_shapes=[pltpu.VMEM((N_GRAPHS, D), jnp.float32)],
)


def kernel(x, edge_index, batch, Wl0, Wr0, bl0, Wl1, Wr1, bl1,
           Wl2, Wr2, bl2, ln_g, ln_b, Wd, bd):
  f32 = jnp.float32
  x_pad = jnp.zeros((N_PAD, D), f32).at[:N_NODES].set(x)
  zeros_tab = jnp.zeros((ROWS_PER_TILE, D), f32)
  ones_blk = jnp.ones((CHUNK, D), f32)

  n_edges = edge_index.shape[1]
  n_fill = E_PAD - n_edges
  fill = jnp.arange(n_fill, dtype=jnp.int32)
  src = jnp.concatenate([edge_index[0], fill % N_NODES])
  dst = jnp.concatenate([edge_index[1], N_NODES + fill % (N_PAD - N_NODES)])
  idx = jnp.stack([src.reshape(NC, NS, CPT, CHUNK),
                   dst.reshape(NC, NS, CPT, CHUNK)], axis=3)

  batch2 = jnp.zeros((N_PAD, 1), jnp.int32).at[:N_NODES, 0].set(batch)

  c0 = _sc_cnt(idx, zeros_tab, ones_blk)
  x_dep, _ = lax.optimization_barrier((x_pad, c0))
  p0 = _sc_segsum(x_dep, idx, zeros_tab)
  h = _tc_layer(p0, c0, x_pad, Wl0.T, Wr0.T, bl0[None])
  p1 = _sc_segsum(h, idx, zeros_tab)
  h = _tc_layer(p1, c0, h, Wl1.T, Wr1.T, bl1[None])
  p2 = _sc_segsum(h, idx, zeros_tab)
  h = _tc_layer(p2, c0, h, Wl2.T, Wr2.T, bl2[None])
  return _tc_final(h, batch2, ln_g[None], ln_b[None], Wd.T, bd[None])


# speedup vs baseline: 1.0319x; 1.0319x over previous
import jax
import jax.numpy as jnp
from jax import lax
from jax.experimental import pallas as pl
from jax.experimental.pallas import tpu as pltpu
from jax.experimental.pallas import tpu_sc as plsc

N_NODES = 10000
N_PAD = 10240
D = 128
N_GRAPHS = 64
NC, NS = 2, 16
CHUNK = 128
GRP = 8
CPT = 80
NGRP = CPT // GRP
E_PAD = NC * NS * CPT * CHUNK
ROWS_PER_TILE = N_PAD // NS
BLK = 512
GRID = N_PAD // BLK

_MESH = plsc.VectorSubcoreMesh(core_axis_name="c", subcore_axis_name="s")


def _segsum_body(h_hbm, idx_hbm, zeros_hbm, out_hbm,
                 win_a, win_b, buf_a, buf_b, acc,
                 sem_ia, sem_ib, sem_ga, sem_gb):
  cid = lax.axis_index("c")
  sid = lax.axis_index("s")
  row0 = sid * ROWS_PER_TILE

  pltpu.sync_copy(zeros_hbm.at[pl.ds(row0, ROWS_PER_TILE)],
                  acc.at[pl.ds(row0, ROWS_PER_TILE)])

  pltpu.async_copy(idx_hbm.at[cid, sid, pl.ds(0, GRP)], win_a, sem_ia)
  pltpu.async_copy(idx_hbm.at[cid, sid, pl.ds(GRP, GRP)], win_b, sem_ib)
  plsc.subcore_barrier()

  bufs = (buf_a, buf_b)
  gsems = (sem_ga, sem_gb)

  def group(g, win, sem_i):
    pltpu.make_async_copy(idx_hbm.at[cid, sid, pl.ds(g * GRP, GRP)],
                          win, sem_i).wait()
    pltpu.async_copy(h_hbm.at[win.at[0, 0]], bufs[0], gsems[0])
    for k in range(GRP):
      if k + 1 < GRP:
        pltpu.async_copy(h_hbm.at[win.at[k + 1, 0]],
                         bufs[(k + 1) % 2], gsems[(k + 1) % 2])
      pltpu.make_async_copy(h_hbm.at[win.at[k, 0]],
                            bufs[k % 2], gsems[k % 2]).wait()
      pltpu.sync_copy(bufs[k % 2], acc.at[win.at[k, 1]], add=True)

    @pl.when(g + 2 < NGRP)
    def _():
      pltpu.async_copy(idx_hbm.at[cid, sid, pl.ds((g + 2) * GRP, GRP)],
                       win, sem_i)

  def pair(j, carry):
    group(2 * j, win_a, sem_ia)
    group(2 * j + 1, win_b, sem_ib)
    return carry

  lax.fori_loop(0, NGRP // 2, pair, 0)
  plsc.subcore_barrier()

  pltpu.sync_copy(acc.at[pl.ds(row0, ROWS_PER_TILE)],
                  out_hbm.at[cid, pl.ds(row0, ROWS_PER_TILE)])


_sc_segsum = pl.kernel(
    _segsum_body,
    out_type=jax.ShapeDtypeStruct((NC, N_PAD, D), jnp.float32),
    mesh=_MESH,
    scratch_types=[
        pltpu.VMEM((GRP, 2, CHUNK), jnp.int32),
        pltpu.VMEM((GRP, 2, CHUNK), jnp.int32),
        pltpu.VMEM((CHUNK, D), jnp.float32),
        pltpu.VMEM((CHUNK, D), jnp.float32),
        pltpu.VMEM_SHARED((N_PAD, D), jnp.float32),
        pltpu.SemaphoreType.DMA,
        pltpu.SemaphoreType.DMA,
        pltpu.SemaphoreType.DMA,
        pltpu.SemaphoreType.DMA,
    ],
)


def _cnt_body(idx_hbm, zeros_hbm, ones_hbm, out_hbm, win_a, win_b, ones_v,
              acc, sem_ia, sem_ib):
  cid = lax.axis_index("c")
  sid = lax.axis_index("s")
  row0 = sid * ROWS_PER_TILE

  pltpu.sync_copy(ones_hbm, ones_v)
  pltpu.sync_copy(zeros_hbm.at[pl.ds(row0, ROWS_PER_TILE)],
                  acc.at[pl.ds(row0, ROWS_PER_TILE)])
  pltpu.async_copy(idx_hbm.at[cid, sid, pl.ds(0, GRP)], win_a, sem_ia)
  pltpu.async_copy(idx_hbm.at[cid, sid, pl.ds(GRP, GRP)], win_b, sem_ib)
  plsc.subcore_barrier()

  def group(g, win, sem_i):
    pltpu.make_async_copy(idx_hbm.at[cid, sid, pl.ds(g * GRP, GRP)],
                          win, sem_i).wait()
    for k in range(GRP):
      pltpu.sync_copy(ones_v, acc.at[win.at[k, 1]], add=True)

    @pl.when(g + 2 < NGRP)
    def _():
      pltpu.async_copy(idx_hbm.at[cid, sid, pl.ds((g + 2) * GRP, GRP)],
                       win, sem_i)

  def pair(j, carry):
    group(2 * j, win_a, sem_ia)
    group(2 * j + 1, win_b, sem_ib)
    return carry

  lax.fori_loop(0, NGRP // 2, pair, 0)
  plsc.subcore_barrier()
  pltpu.sync_copy(acc.at[pl.ds(row0, ROWS_PER_TILE)],
                  out_hbm.at[cid, pl.ds(row0, ROWS_PER_TILE)])


_sc_cnt = pl.kernel(
    _cnt_body,
    out_type=jax.ShapeDtypeStruct((NC, N_PAD, D), jnp.float32),
    mesh=_MESH,
    scratch_types=[
        pltpu.VMEM((GRP, 2, CHUNK), jnp.int32),
        pltpu.VMEM((GRP, 2, CHUNK), jnp.int32),
        pltpu.VMEM((CHUNK, D), jnp.float32),
        pltpu.VMEM_SHARED((N_PAD, D), jnp.float32),
        pltpu.SemaphoreType.DMA,
        pltpu.SemaphoreType.DMA,
    ],
)


def _compute_h(p_ref, inv, x_ref, wl_ref, wr_ref, bl_ref, i):
  msg = p_ref[0] + p_ref[1]
  h = inv * jnp.dot(msg, wl_ref[...], preferred_element_type=jnp.float32)
  h = h + bl_ref[...] + jnp.dot(x_ref[...], wr_ref[...],
                                preferred_element_type=jnp.float32)
  h = jnp.maximum(h, 0.0)
  rowid = i * BLK + lax.broadcasted_iota(jnp.int32, (BLK, 1), 0)
  return jnp.where(rowid < N_NODES, h, 0.0)


def _layer0_body(p_ref, c_ref, x_ref, wl_ref, wr_ref, bl_ref,
                 o_ref, inv_ref):
  i = pl.program_id(0)
  cnt = c_ref[0][:, 0:1] + c_ref[1][:, 0:1]
  inv = 1.0 / jnp.maximum(cnt, 1.0)
  inv_ref[...] = inv
  o_ref[...] = _compute_h(p_ref, inv, x_ref, wl_ref, wr_ref, bl_ref, i)


_tc_layer0 = pl.pallas_call(
    _layer0_body,
    grid=(GRID,),
    in_specs=[
        pl.BlockSpec((NC, BLK, D), lambda i: (0, i, 0)),
        pl.BlockSpec((NC, BLK, D), lambda i: (0, i, 0)),
        pl.BlockSpec((BLK, D), lambda i: (i, 0)),
        pl.BlockSpec((D, D), lambda i: (0, 0)),
        pl.BlockSpec((D, D), lambda i: (0, 0)),
        pl.BlockSpec((1, D), lambda i: (0, 0)),
    ],
    out_specs=[pl.BlockSpec((BLK, D), lambda i: (i, 0)),
               pl.BlockSpec((BLK, 1), lambda i: (i, 0))],
    out_shape=[jax.ShapeDtypeStruct((N_PAD, D), jnp.float32),
               jax.ShapeDtypeStruct((N_PAD, 1), jnp.float32)],
)


def _layer1_body(p_ref, inv_ref, x_ref, wl_ref, wr_ref, bl_ref, o_ref):
  i = pl.program_id(0)
  o_ref[...] = _compute_h(p_ref, inv_ref[...], x_ref, wl_ref, wr_ref,
                          bl_ref, i)


_tc_layer1 = pl.pallas_call(
    _layer1_body,
    grid=(GRID,),
    in_specs=[
        pl.BlockSpec((NC, BLK, D), lambda i: (0, i, 0)),
        pl.BlockSpec((BLK, 1), lambda i: (i, 0)),
        pl.BlockSpec((BLK, D), lambda i: (i, 0)),
        pl.BlockSpec((D, D), lambda i: (0, 0)),
        pl.BlockSpec((D, D), lambda i: (0, 0)),
        pl.BlockSpec((1, D), lambda i: (0, 0)),
    ],
    out_specs=pl.BlockSpec((BLK, D), lambda i: (i, 0)),
    out_shape=jax.ShapeDtypeStruct((N_PAD, D), jnp.float32),
)


def _layer2_final_body(p_ref, inv_ref, x_ref, wl_ref, wr_ref, bl_ref,
                       b_ref, g_ref, beta_ref, wd_ref, bd_ref, o_ref, pool):
  i = pl.program_id(0)

  @pl.when(i == 0)
  def _():
    pool[...] = jnp.zeros_like(pool)

  h = _compute_h(p_ref, inv_ref[...], x_ref, wl_ref, wr_ref, bl_ref, i)
  b = b_ref[...]
  onehot = (b == lax.broadcasted_iota(jnp.int32, (1, N_GRAPHS), 1)
            ).astype(jnp.float32)
  pool[...] += lax.dot_general(onehot, h, (((0,), (0,)), ((), ())),
                               preferred_element_type=jnp.float32)

  @pl.when(i == GRID - 1)
  def _():
    p = pool[...]
    m = jnp.mean(p, axis=1, keepdims=True)
    v = jnp.mean((p - m) * (p - m), axis=1, keepdims=True)
    ln = (p - m) * lax.rsqrt(v + 1e-5) * g_ref[...] + beta_ref[...]
    o_ref[...] = jnp.dot(ln, wd_ref[...],
                         preferred_element_type=jnp.float32) + bd_ref[...]


_tc_layer2_final = pl.pallas_call(
    _layer2_final_body,
    grid=(GRID,),
    in_specs=[
        pl.BlockSpec((NC, BLK, D), lambda i: (0, i, 0)),
        pl.BlockSpec((BLK, 1), lambda i: (i, 0)),
        pl.BlockSpec((BLK, D), lambda i: (i, 0)),
        pl.BlockSpec((D, D), lambda i: (0, 0)),
        pl.BlockSpec((D, D), lambda i: (0, 0)),
        pl.BlockSpec((1, D), lambda i: (0, 0)),
        pl.BlockSpec((BLK, 1), lambda i: (i, 0)),
        pl.BlockSpec((1, D), lambda i: (0, 0)),
        pl.BlockSpec((1, D), lambda i: (0, 0)),
        pl.BlockSpec((D, D), lambda i: (0, 0)),
        pl.BlockSpec((1, D), lambda i: (0, 0)),
    ],
    out_specs=pl.BlockSpec((N_GRAPHS, D), lambda i: (0, 0)),
    out_shape=jax.ShapeDtypeStruct((N_GRAPHS, D), jnp.float32),
    scratch_shapes=[pltpu.VMEM((N_GRAPHS, D), jnp.float32)],
)


def kernel(x, edge_index, batch, Wl0, Wr0, bl0, Wl1, Wr1, bl1,
           Wl2, Wr2, bl2, ln_g, ln_b, Wd, bd):
  f32 = jnp.float32
  x_pad = jnp.zeros((N_PAD, D), f32).at[:N_NODES].set(x)
  zeros_tab = jnp.zeros((N_PAD, D), f32)
  ones_blk = jnp.ones((CHUNK, D), f32)

  n_edges = edge_index.shape[1]
  n_fill = E_PAD - n_edges
  fill = jnp.arange(n_fill, dtype=jnp.int32)
  src = jnp.concatenate([edge_index[0], fill % N_NODES])
  dst = jnp.concatenate([edge_index[1], N_NODES + fill % (N_PAD - N_NODES)])
  idx = jnp.stack([src.reshape(NC, NS, CPT, CHUNK),
                   dst.reshape(NC, NS, CPT, CHUNK)], axis=3)

  batch2 = jnp.zeros((N_PAD, 1), jnp.int32).at[:N_NODES, 0].set(batch)

  c0 = _sc_cnt(idx, zeros_tab, ones_blk)
  x_dep, _ = lax.optimization_barrier((x_pad, c0))
  p0 = _sc_segsum(x_dep, idx, zeros_tab)
  h, inv = _tc_layer0(p0, c0, x_pad, Wl0.T, Wr0.T, bl0[None])
  p1 = _sc_segsum(h, idx, zeros_tab)
  h = _tc_layer1(p1, inv, h, Wl1.T, Wr1.T, bl1[None])
  p2 = _sc_segsum(h, idx, zeros_tab)
  return _tc_layer2_final(p2, inv, h, Wl2.T, Wr2.T, bl2[None], batch2,
                          ln_g[None], ln_b[None], Wd.T, bd[None])


# speedup vs baseline: 1.0711x; 1.0379x over previous
import jax
import jax.numpy as jnp
from jax import lax
from jax.experimental import pallas as pl
from jax.experimental.pallas import tpu as pltpu
from jax.experimental.pallas import tpu_sc as plsc

N_NODES = 10000
N_PAD = 10240
D = 128
N_GRAPHS = 64
NC, NS = 2, 16
CHUNK = 128
GRP = 16
CPT = 80
NGRP = CPT // GRP
E_PAD = NC * NS * CPT * CHUNK
ROWS_PER_TILE = N_PAD // NS
BLK = 512
GRID = N_PAD // BLK

_MESH = plsc.VectorSubcoreMesh(core_axis_name="c", subcore_axis_name="s")


def _segsum_body(h_hbm, idx_hbm, zeros_hbm, out_hbm,
                 win_a, win_b, buf_a, buf_b, acc,
                 sem_ia, sem_ib, sem_ga, sem_gb):
  cid = lax.axis_index("c")
  sid = lax.axis_index("s")
  row0 = sid * ROWS_PER_TILE

  pltpu.sync_copy(zeros_hbm.at[pl.ds(row0, ROWS_PER_TILE)],
                  acc.at[pl.ds(row0, ROWS_PER_TILE)])

  pltpu.async_copy(idx_hbm.at[cid, sid, pl.ds(0, GRP)], win_a, sem_ia)
  pltpu.async_copy(idx_hbm.at[cid, sid, pl.ds(GRP, GRP)], win_b, sem_ib)
  plsc.subcore_barrier()

  bufs = (buf_a, buf_b)
  gsems = (sem_ga, sem_gb)

  def group(g, win, sem_i):
    pltpu.make_async_copy(idx_hbm.at[cid, sid, pl.ds(g * GRP, GRP)],
                          win, sem_i).wait()
    pltpu.async_copy(h_hbm.at[win.at[0, 0]], bufs[0], gsems[0])
    for k in range(GRP):
      if k + 1 < GRP:
        pltpu.async_copy(h_hbm.at[win.at[k + 1, 0]],
                         bufs[(k + 1) % 2], gsems[(k + 1) % 2])
      pltpu.make_async_copy(h_hbm.at[win.at[k, 0]],
                            bufs[k % 2], gsems[k % 2]).wait()
      pltpu.sync_copy(bufs[k % 2], acc.at[win.at[k, 1]], add=True)

    @pl.when(g + 2 < NGRP)
    def _():
      pltpu.async_copy(idx_hbm.at[cid, sid, pl.ds((g + 2) * GRP, GRP)],
                       win, sem_i)

  def pair(j, carry):
    group(2 * j, win_a, sem_ia)
    group(2 * j + 1, win_b, sem_ib)
    return carry

  lax.fori_loop(0, NGRP // 2, pair, 0)
  if NGRP % 2:
    group(NGRP - 1, win_a, sem_ia)
  plsc.subcore_barrier()

  pltpu.sync_copy(acc.at[pl.ds(row0, ROWS_PER_TILE)],
                  out_hbm.at[cid, pl.ds(row0, ROWS_PER_TILE)])


_sc_segsum = pl.kernel(
    _segsum_body,
    out_type=jax.ShapeDtypeStruct((NC, N_PAD, D), jnp.float32),
    mesh=_MESH,
    scratch_types=[
        pltpu.VMEM((GRP, 2, CHUNK), jnp.int32),
        pltpu.VMEM((GRP, 2, CHUNK), jnp.int32),
        pltpu.VMEM((CHUNK, D), jnp.float32),
        pltpu.VMEM((CHUNK, D), jnp.float32),
        pltpu.VMEM_SHARED((N_PAD, D), jnp.float32),
        pltpu.SemaphoreType.DMA,
        pltpu.SemaphoreType.DMA,
        pltpu.SemaphoreType.DMA,
        pltpu.SemaphoreType.DMA,
    ],
)


def _cnt_body(idx_hbm, zeros_hbm, ones_hbm, out_hbm, win_a, win_b, ones_v,
              acc, sem_ia, sem_ib):
  cid = lax.axis_index("c")
  sid = lax.axis_index("s")
  row0 = sid * ROWS_PER_TILE

  pltpu.sync_copy(ones_hbm, ones_v)
  pltpu.sync_copy(zeros_hbm.at[pl.ds(row0, ROWS_PER_TILE)],
                  acc.at[pl.ds(row0, ROWS_PER_TILE)])
  pltpu.async_copy(idx_hbm.at[cid, sid, pl.ds(0, GRP)], win_a, sem_ia)
  pltpu.async_copy(idx_hbm.at[cid, sid, pl.ds(GRP, GRP)], win_b, sem_ib)
  plsc.subcore_barrier()

  def group(g, win, sem_i):
    pltpu.make_async_copy(idx_hbm.at[cid, sid, pl.ds(g * GRP, GRP)],
                          win, sem_i).wait()
    for k in range(GRP):
      pltpu.sync_copy(ones_v, acc.at[win.at[k, 1]], add=True)

    @pl.when(g + 2 < NGRP)
    def _():
      pltpu.async_copy(idx_hbm.at[cid, sid, pl.ds((g + 2) * GRP, GRP)],
                       win, sem_i)

  def pair(j, carry):
    group(2 * j, win_a, sem_ia)
    group(2 * j + 1, win_b, sem_ib)
    return carry

  lax.fori_loop(0, NGRP // 2, pair, 0)
  if NGRP % 2:
    group(NGRP - 1, win_a, sem_ia)
  plsc.subcore_barrier()
  pltpu.sync_copy(acc.at[pl.ds(row0, ROWS_PER_TILE)],
                  out_hbm.at[cid, pl.ds(row0, ROWS_PER_TILE)])


_sc_cnt = pl.kernel(
    _cnt_body,
    out_type=jax.ShapeDtypeStruct((NC, N_PAD, D), jnp.float32),
    mesh=_MESH,
    scratch_types=[
        pltpu.VMEM((GRP, 2, CHUNK), jnp.int32),
        pltpu.VMEM((GRP, 2, CHUNK), jnp.int32),
        pltpu.VMEM((CHUNK, D), jnp.float32),
        pltpu.VMEM_SHARED((N_PAD, D), jnp.float32),
        pltpu.SemaphoreType.DMA,
        pltpu.SemaphoreType.DMA,
    ],
)


def _compute_h(p_ref, inv, x_ref, wl_ref, wr_ref, bl_ref, i):
  msg = p_ref[0] + p_ref[1]
  h = inv * jnp.dot(msg, wl_ref[...], preferred_element_type=jnp.float32)
  h = h + bl_ref[...] + jnp.dot(x_ref[...], wr_ref[...],
                                preferred_element_type=jnp.float32)
  h = jnp.maximum(h, 0.0)
  rowid = i * BLK + lax.broadcasted_iota(jnp.int32, (BLK, 1), 0)
  return jnp.where(rowid < N_NODES, h, 0.0)


def _layer0_body(p_ref, c_ref, x_ref, wl_ref, wr_ref, bl_ref,
                 o_ref, inv_ref):
  i = pl.program_id(0)
  cnt = c_ref[0][:, 0:1] + c_ref[1][:, 0:1]
  inv = 1.0 / jnp.maximum(cnt, 1.0)
  inv_ref[...] = inv
  o_ref[...] = _compute_h(p_ref, inv, x_ref, wl_ref, wr_ref, bl_ref, i)


_tc_layer0 = pl.pallas_call(
    _layer0_body,
    grid=(GRID,),
    in_specs=[
        pl.BlockSpec((NC, BLK, D), lambda i: (0, i, 0)),
        pl.BlockSpec((NC, BLK, D), lambda i: (0, i, 0)),
        pl.BlockSpec((BLK, D), lambda i: (i, 0)),
        pl.BlockSpec((D, D), lambda i: (0, 0)),
        pl.BlockSpec((D, D), lambda i: (0, 0)),
        pl.BlockSpec((1, D), lambda i: (0, 0)),
    ],
    out_specs=[pl.BlockSpec((BLK, D), lambda i: (i, 0)),
               pl.BlockSpec((BLK, 1), lambda i: (i, 0))],
    out_shape=[jax.ShapeDtypeStruct((N_PAD, D), jnp.float32),
               jax.ShapeDtypeStruct((N_PAD, 1), jnp.float32)],
)


def _layer1_body(p_ref, inv_ref, x_ref, wl_ref, wr_ref, bl_ref, o_ref):
  i = pl.program_id(0)
  o_ref[...] = _compute_h(p_ref, inv_ref[...], x_ref, wl_ref, wr_ref,
                          bl_ref, i)


_tc_layer1 = pl.pallas_call(
    _layer1_body,
    grid=(GRID,),
    in_specs=[
        pl.BlockSpec((NC, BLK, D), lambda i: (0, i, 0)),
        pl.BlockSpec((BLK, 1), lambda i: (i, 0)),
        pl.BlockSpec((BLK, D), lambda i: (i, 0)),
        pl.BlockSpec((D, D), lambda i: (0, 0)),
        pl.BlockSpec((D, D), lambda i: (0, 0)),
        pl.BlockSpec((1, D), lambda i: (0, 0)),
    ],
    out_specs=pl.BlockSpec((BLK, D), lambda i: (i, 0)),
    out_shape=jax.ShapeDtypeStruct((N_PAD, D), jnp.float32),
)


def _layer2_final_body(p_ref, inv_ref, x_ref, wl_ref, wr_ref, bl_ref,
                       b_ref, g_ref, beta_ref, wd_ref, bd_ref, o_ref, pool):
  i = pl.program_id(0)

  @pl.when(i == 0)
  def _():
    pool[...] = jnp.zeros_like(pool)

  h = _compute_h(p_ref, inv_ref[...], x_ref, wl_ref, wr_ref, bl_ref, i)
  b = b_ref[...]
  onehot = (b == lax.broadcasted_iota(jnp.int32, (1, N_GRAPHS), 1)
            ).astype(jnp.float32)
  pool[...] += lax.dot_general(onehot, h, (((0,), (0,)), ((), ())),
                               preferred_element_type=jnp.float32)

  @pl.when(i == GRID - 1)
  def _():
    p = pool[...]
    m = jnp.mean(p, axis=1, keepdims=True)
    v = jnp.mean((p - m) * (p - m), axis=1, keepdims=True)
    ln = (p - m) * lax.rsqrt(v + 1e-5) * g_ref[...] + beta_ref[...]
    o_ref[...] = jnp.dot(ln, wd_ref[...],
                         preferred_element_type=jnp.float32) + bd_ref[...]


_tc_layer2_final = pl.pallas_call(
    _layer2_final_body,
    grid=(GRID,),
    in_specs=[
        pl.BlockSpec((NC, BLK, D), lambda i: (0, i, 0)),
        pl.BlockSpec((BLK, 1), lambda i: (i, 0)),
        pl.BlockSpec((BLK, D), lambda i: (i, 0)),
        pl.BlockSpec((D, D), lambda i: (0, 0)),
        pl.BlockSpec((D, D), lambda i: (0, 0)),
        pl.BlockSpec((1, D), lambda i: (0, 0)),
        pl.BlockSpec((BLK, 1), lambda i: (i, 0)),
        pl.BlockSpec((1, D), lambda i: (0, 0)),
        pl.BlockSpec((1, D), lambda i: (0, 0)),
        pl.BlockSpec((D, D), lambda i: (0, 0)),
        pl.BlockSpec((1, D), lambda i: (0, 0)),
    ],
    out_specs=pl.BlockSpec((N_GRAPHS, D), lambda i: (0, 0)),
    out_shape=jax.ShapeDtypeStruct((N_GRAPHS, D), jnp.float32),
    scratch_shapes=[pltpu.VMEM((N_GRAPHS, D), jnp.float32)],
)


def kernel(x, edge_index, batch, Wl0, Wr0, bl0, Wl1, Wr1, bl1,
           Wl2, Wr2, bl2, ln_g, ln_b, Wd, bd):
  f32 = jnp.float32
  x_pad = jnp.zeros((N_PAD, D), f32).at[:N_NODES].set(x)
  zeros_tab = jnp.zeros((N_PAD, D), f32)
  ones_blk = jnp.ones((CHUNK, D), f32)

  n_edges = edge_index.shape[1]
  n_fill = E_PAD - n_edges
  fill = jnp.arange(n_fill, dtype=jnp.int32)
  src = jnp.concatenate([edge_index[0], fill % N_NODES])
  dst = jnp.concatenate([edge_index[1], N_NODES + fill % (N_PAD - N_NODES)])
  idx = jnp.stack([src.reshape(NC, NS, CPT, CHUNK),
                   dst.reshape(NC, NS, CPT, CHUNK)], axis=3)

  batch2 = jnp.zeros((N_PAD, 1), jnp.int32).at[:N_NODES, 0].set(batch)

  c0 = _sc_cnt(idx, zeros_tab, ones_blk)
  x_dep, _ = lax.optimization_barrier((x_pad, c0))
  p0 = _sc_segsum(x_dep, idx, zeros_tab)
  h, inv = _tc_layer0(p0, c0, x_pad, Wl0.T, Wr0.T, bl0[None])
  p1 = _sc_segsum(h, idx, zeros_tab)
  h = _tc_layer1(p1, inv, h, Wl1.T, Wr1.T, bl1[None])
  p2 = _sc_segsum(h, idx, zeros_tab)
  return _tc_layer2_final(p2, inv, h, Wl2.T, Wr2.T, bl2[None], batch2,
                          ln_g[None], ln_b[None], Wd.T, bd[None])


# speedup vs baseline: 1.0763x; 1.0049x over previous
import jax
import jax.numpy as jnp
from jax import lax
from jax.experimental import pallas as pl
from jax.experimental.pallas import tpu as pltpu
from jax.experimental.pallas import tpu_sc as plsc

N_NODES = 10000
N_PAD = 10240
D = 128
N_GRAPHS = 64
NC, NS = 2, 16
CHUNK = 128
GRP = 16
CPT = 80
NGRP = CPT // GRP
E_PAD = NC * NS * CPT * CHUNK
ROWS_PER_TILE = N_PAD // NS
BLK = 512
GRID = N_PAD // BLK

_MESH = plsc.VectorSubcoreMesh(core_axis_name="c", subcore_axis_name="s")


def _segsum_body(h_hbm, src_hbm, dst_hbm, zeros_hbm, out_hbm,
                 wsa, wsb, wda, wdb, buf_a, buf_b, acc,
                 sem_ia, sem_ib, sem_ga, sem_gb):
  cid = lax.axis_index("c")
  sid = lax.axis_index("s")
  row0 = sid * ROWS_PER_TILE

  pltpu.sync_copy(zeros_hbm.at[pl.ds(row0, ROWS_PER_TILE)],
                  acc.at[pl.ds(row0, ROWS_PER_TILE)])

  def fetch(g, ws, wd, sem_i):
    pltpu.async_copy(src_hbm.at[cid, sid, pl.ds(g * GRP, GRP)], ws, sem_i)
    pltpu.async_copy(dst_hbm.at[cid, sid, pl.ds(g * GRP, GRP)], wd, sem_i)

  def fetch_wait(g, ws, wd, sem_i):
    pltpu.make_async_copy(src_hbm.at[cid, sid, pl.ds(g * GRP, GRP)],
                          ws, sem_i).wait()
    pltpu.make_async_copy(dst_hbm.at[cid, sid, pl.ds(g * GRP, GRP)],
                          wd, sem_i).wait()

  fetch(0, wsa, wda, sem_ia)
  fetch(1, wsb, wdb, sem_ib)
  plsc.subcore_barrier()

  bufs = (buf_a, buf_b)
  gsems = (sem_ga, sem_gb)

  def group(g, ws, wd, sem_i):
    fetch_wait(g, ws, wd, sem_i)
    pltpu.async_copy(h_hbm.at[ws.at[0]], bufs[0], gsems[0])
    for k in range(GRP):
      if k + 1 < GRP:
        pltpu.async_copy(h_hbm.at[ws.at[k + 1]],
                         bufs[(k + 1) % 2], gsems[(k + 1) % 2])
      pltpu.make_async_copy(h_hbm.at[ws.at[k]],
                            bufs[k % 2], gsems[k % 2]).wait()
      pltpu.sync_copy(bufs[k % 2], acc.at[wd.at[k]], add=True)

    @pl.when(g + 2 < NGRP)
    def _():
      fetch(g + 2, ws, wd, sem_i)

  def pair(j, carry):
    group(2 * j, wsa, wda, sem_ia)
    group(2 * j + 1, wsb, wdb, sem_ib)
    return carry

  lax.fori_loop(0, NGRP // 2, pair, 0)
  if NGRP % 2:
    group(NGRP - 1, wsa, wda, sem_ia)
  plsc.subcore_barrier()

  pltpu.sync_copy(acc.at[pl.ds(row0, ROWS_PER_TILE)],
                  out_hbm.at[cid, pl.ds(row0, ROWS_PER_TILE)])


_sc_segsum = pl.kernel(
    _segsum_body,
    out_type=jax.ShapeDtypeStruct((NC, N_PAD, D), jnp.float32),
    mesh=_MESH,
    scratch_types=[
        pltpu.VMEM((GRP, CHUNK), jnp.int32),
        pltpu.VMEM((GRP, CHUNK), jnp.int32),
        pltpu.VMEM((GRP, CHUNK), jnp.int32),
        pltpu.VMEM((GRP, CHUNK), jnp.int32),
        pltpu.VMEM((CHUNK, D), jnp.float32),
        pltpu.VMEM((CHUNK, D), jnp.float32),
        pltpu.VMEM_SHARED((N_PAD, D), jnp.float32),
        pltpu.SemaphoreType.DMA,
        pltpu.SemaphoreType.DMA,
        pltpu.SemaphoreType.DMA,
        pltpu.SemaphoreType.DMA,
    ],
)


def _cnt_body(dst_hbm, zeros_hbm, ones_hbm, out_hbm, wda, wdb, ones_v,
              acc, sem_ia, sem_ib):
  cid = lax.axis_index("c")
  sid = lax.axis_index("s")
  row0 = sid * ROWS_PER_TILE

  pltpu.sync_copy(ones_hbm, ones_v)
  pltpu.sync_copy(zeros_hbm.at[pl.ds(row0, ROWS_PER_TILE)],
                  acc.at[pl.ds(row0, ROWS_PER_TILE)])
  pltpu.async_copy(dst_hbm.at[cid, sid, pl.ds(0, GRP)], wda, sem_ia)
  pltpu.async_copy(dst_hbm.at[cid, sid, pl.ds(GRP, GRP)], wdb, sem_ib)
  plsc.subcore_barrier()

  def group(g, wd, sem_i):
    pltpu.make_async_copy(dst_hbm.at[cid, sid, pl.ds(g * GRP, GRP)],
                          wd, sem_i).wait()
    for k in range(GRP):
      pltpu.sync_copy(ones_v, acc.at[wd.at[k]], add=True)

    @pl.when(g + 2 < NGRP)
    def _():
      pltpu.async_copy(dst_hbm.at[cid, sid, pl.ds((g + 2) * GRP, GRP)],
                       wd, sem_i)

  def pair(j, carry):
    group(2 * j, wda, sem_ia)
    group(2 * j + 1, wdb, sem_ib)
    return carry

  lax.fori_loop(0, NGRP // 2, pair, 0)
  if NGRP % 2:
    group(NGRP - 1, wda, sem_ia)
  plsc.subcore_barrier()
  pltpu.sync_copy(acc.at[pl.ds(row0, ROWS_PER_TILE)],
                  out_hbm.at[cid, pl.ds(row0, ROWS_PER_TILE)])


_sc_cnt = pl.kernel(
    _cnt_body,
    out_type=jax.ShapeDtypeStruct((NC, N_PAD, D), jnp.float32),
    mesh=_MESH,
    scratch_types=[
        pltpu.VMEM((GRP, CHUNK), jnp.int32),
        pltpu.VMEM((GRP, CHUNK), jnp.int32),
        pltpu.VMEM((CHUNK, D), jnp.float32),
        pltpu.VMEM_SHARED((N_PAD, D), jnp.float32),
        pltpu.SemaphoreType.DMA,
        pltpu.SemaphoreType.DMA,
    ],
)


def _compute_h(p_ref, inv, x_ref, wl_ref, wr_ref, bl_ref, i):
  msg = p_ref[0] + p_ref[1]
  h = inv * jnp.dot(msg, wl_ref[...], preferred_element_type=jnp.float32)
  h = h + bl_ref[...] + jnp.dot(x_ref[...], wr_ref[...],
                                preferred_element_type=jnp.float32)
  h = jnp.maximum(h, 0.0)
  rowid = i * BLK + lax.broadcasted_iota(jnp.int32, (BLK, 1), 0)
  return jnp.where(rowid < N_NODES, h, 0.0)


def _layer0_body(p_ref, c_ref, x_ref, wl_ref, wr_ref, bl_ref,
                 o_ref, inv_ref):
  i = pl.program_id(0)
  cnt = c_ref[0][:, 0:1] + c_ref[1][:, 0:1]
  inv = 1.0 / jnp.maximum(cnt, 1.0)
  inv_ref[...] = inv
  o_ref[...] = _compute_h(p_ref, inv, x_ref, wl_ref, wr_ref, bl_ref, i)


_tc_layer0 = pl.pallas_call(
    _layer0_body,
    grid=(GRID,),
    in_specs=[
        pl.BlockSpec((NC, BLK, D), lambda i: (0, i, 0)),
        pl.BlockSpec((NC, BLK, D), lambda i: (0, i, 0)),
        pl.BlockSpec((BLK, D), lambda i: (i, 0)),
        pl.BlockSpec((D, D), lambda i: (0, 0)),
        pl.BlockSpec((D, D), lambda i: (0, 0)),
        pl.BlockSpec((1, D), lambda i: (0, 0)),
    ],
    out_specs=[pl.BlockSpec((BLK, D), lambda i: (i, 0)),
               pl.BlockSpec((BLK, 1), lambda i: (i, 0))],
    out_shape=[jax.ShapeDtypeStruct((N_PAD, D), jnp.float32),
               jax.ShapeDtypeStruct((N_PAD, 1), jnp.float32)],
)


def _layer1_body(p_ref, inv_ref, x_ref, wl_ref, wr_ref, bl_ref, o_ref):
  i = pl.program_id(0)
  o_ref[...] = _compute_h(p_ref, inv_ref[...], x_ref, wl_ref, wr_ref,
                          bl_ref, i)


_tc_layer1 = pl.pallas_call(
    _layer1_body,
    grid=(GRID,),
    in_specs=[
        pl.BlockSpec((NC, BLK, D), lambda i: (0, i, 0)),
        pl.BlockSpec((BLK, 1), lambda i: (i, 0)),
        pl.BlockSpec((BLK, D), lambda i: (i, 0)),
        pl.BlockSpec((D, D), lambda i: (0, 0)),
        pl.BlockSpec((D, D), lambda i: (0, 0)),
        pl.BlockSpec((1, D), lambda i: (0, 0)),
    ],
    out_specs=pl.BlockSpec((BLK, D), lambda i: (i, 0)),
    out_shape=jax.ShapeDtypeStruct((N_PAD, D), jnp.float32),
)


def _layer2_final_body(p_ref, inv_ref, x_ref, wl_ref, wr_ref, bl_ref,
                       b_ref, g_ref, beta_ref, wd_ref, bd_ref, o_ref, pool):
  i = pl.program_id(0)

  @pl.when(i == 0)
  def _():
    pool[...] = jnp.zeros_like(pool)

  h = _compute_h(p_ref, inv_ref[...], x_ref, wl_ref, wr_ref, bl_ref, i)
  b = b_ref[...]
  onehot = (b == lax.broadcasted_iota(jnp.int32, (1, N_GRAPHS), 1)
            ).astype(jnp.float32)
  pool[...] += lax.dot_general(onehot, h, (((0,), (0,)), ((), ())),
                               preferred_element_type=jnp.float32)

  @pl.when(i == GRID - 1)
  def _():
    p = pool[...]
    m = jnp.mean(p, axis=1, keepdims=True)
    v = jnp.mean((p - m) * (p - m), axis=1, keepdims=True)
    ln = (p - m) * lax.rsqrt(v + 1e-5) * g_ref[...] + beta_ref[...]
    o_ref[...] = jnp.dot(ln, wd_ref[...],
                         preferred_element_type=jnp.float32) + bd_ref[...]


_tc_layer2_final = pl.pallas_call(
    _layer2_final_body,
    grid=(GRID,),
    in_specs=[
        pl.BlockSpec((NC, BLK, D), lambda i: (0, i, 0)),
        pl.BlockSpec((BLK, 1), lambda i: (i, 0)),
        pl.BlockSpec((BLK, D), lambda i: (i, 0)),
        pl.BlockSpec((D, D), lambda i: (0, 0)),
        pl.BlockSpec((D, D), lambda i: (0, 0)),
        pl.BlockSpec((1, D), lambda i: (0, 0)),
        pl.BlockSpec((BLK, 1), lambda i: (i, 0)),
        pl.BlockSpec((1, D), lambda i: (0, 0)),
        pl.BlockSpec((1, D), lambda i: (0, 0)),
        pl.BlockSpec((D, D), lambda i: (0, 0)),
        pl.BlockSpec((1, D), lambda i: (0, 0)),
    ],
    out_specs=pl.BlockSpec((N_GRAPHS, D), lambda i: (0, 0)),
    out_shape=jax.ShapeDtypeStruct((N_GRAPHS, D), jnp.float32),
    scratch_shapes=[pltpu.VMEM((N_GRAPHS, D), jnp.float32)],
)


def kernel(x, edge_index, batch, Wl0, Wr0, bl0, Wl1, Wr1, bl1,
           Wl2, Wr2, bl2, ln_g, ln_b, Wd, bd):
  f32 = jnp.float32
  x_pad = jnp.zeros((N_PAD, D), f32).at[:N_NODES].set(x)
  zeros_tab = jnp.zeros((N_PAD, D), f32)
  ones_blk = jnp.ones((CHUNK, D), f32)

  n_edges = edge_index.shape[1]
  n_fill = E_PAD - n_edges
  fill = jnp.arange(n_fill, dtype=jnp.int32)
  src = jnp.concatenate([edge_index[0], fill % N_NODES])
  dst = jnp.concatenate([edge_index[1], N_NODES + fill % (N_PAD - N_NODES)])
  src_r = src.reshape(NC, NS, CPT, CHUNK)
  dst_r = dst.reshape(NC, NS, CPT, CHUNK)

  batch2 = jnp.zeros((N_PAD, 1), jnp.int32).at[:N_NODES, 0].set(batch)

  c0 = _sc_cnt(dst_r, zeros_tab, ones_blk)
  x_dep, _ = lax.optimization_barrier((x_pad, c0))
  p0 = _sc_segsum(x_dep, src_r, dst_r, zeros_tab)
  h, inv = _tc_layer0(p0, c0, x_pad, Wl0.T, Wr0.T, bl0[None])
  p1 = _sc_segsum(h, src_r, dst_r, zeros_tab)
  h = _tc_layer1(p1, inv, h, Wl1.T, Wr1.T, bl1[None])
  p2 = _sc_segsum(h, src_r, dst_r, zeros_tab)
  return _tc_layer2_final(p2, inv, h, Wl2.T, Wr2.T, bl2[None], batch2,
                          ln_g[None], ln_b[None], Wd.T, bd[None])
